# baseline (speedup 1.0000x reference)
import functools

import jax
import jax.numpy as jnp
from jax import lax
from jax.experimental import pallas as pl
from jax.experimental.pallas import tpu as pltpu
from jax.experimental.pallas import tpu_sc as plsc

_Q = 1024
_MEM = 100000
_D = 128
_K = 8
_BM = 2048
_NB = (_MEM + _BM - 1) // _BM
_NEG = float("-inf")


def _topk_body(q_ref, k_ref, ti_ref, tv_s, ti_s, qn_s, sim_s):
    j = pl.program_id(0)

    @pl.when(j == 0)
    def _():
        tv_s[...] = jnp.full((_Q, _K), _NEG, jnp.float32)
        ti_s[...] = jnp.zeros((_Q, _K), jnp.int32)
        q = q_ref[...]
        qn_s[...] = q / jnp.maximum(
            jnp.sqrt(jnp.sum(q * q, axis=1, keepdims=True)), 1e-12)

    kb = k_ref[...]
    kn = kb / jnp.maximum(
        jnp.sqrt(jnp.sum(kb * kb, axis=1, keepdims=True)), 1e-12)
    sim = lax.dot_general(qn_s[...], kn, (((1,), (1,)), ((), ())),
                          preferred_element_type=jnp.float32)
    col = lax.broadcasted_iota(jnp.int32, (_Q, _BM), 1) + j * _BM

    @pl.when(j == _NB - 1)
    def _():
        sim_s[...] = jnp.where(col < _MEM, sim, _NEG)

    @pl.when(j < _NB - 1)
    def _():
        sim_s[...] = sim

    big = jnp.int32(2**31 - 1)
    _G = 4
    qg = _Q // _G
    slot = lax.broadcasted_iota(jnp.int32, (qg, _K), 1)
    colg = col[:qg]

    def cond(carry):
        k, m, t8 = carry
        return jnp.logical_and(k < _K, jnp.any(m > t8))

    def body(carry):
        k, m, _ = carry
        m_new = []
        t8_new = []
        for g in range(_G):
            r0 = g * qg
            s = sim_s[r0:r0 + qg, :]
            mg = m[r0:r0 + qg]
            im = jnp.min(jnp.where(s == mg, colg, big), axis=1,
                         keepdims=True)
            masked = jnp.where(colg == im, _NEG, s)
            sim_s[r0:r0 + qg, :] = masked
            tv = tv_s[r0:r0 + qg, :]
            ti = ti_s[r0:r0 + qg, :]
            ahead = jnp.logical_or(
                tv > mg, jnp.logical_and(tv == mg, ti < im))
            pos = jnp.sum(ahead.astype(jnp.int32), axis=1, keepdims=True)
            tv_sh = jnp.concatenate([tv[:, :1], tv[:, :-1]], axis=1)
            ti_sh = jnp.concatenate([ti[:, :1], ti[:, :-1]], axis=1)
            keep = slot < pos
            at = slot == pos
            tv_new = jnp.where(keep, tv, jnp.where(at, mg, tv_sh))
            ti_new = jnp.where(keep, ti, jnp.where(at, im, ti_sh))
            tv_s[r0:r0 + qg, :] = tv_new
            ti_s[r0:r0 + qg, :] = ti_new
            m_new.append(jnp.max(masked, axis=1, keepdims=True))
            t8_new.append(tv_new[:, _K - 1:])
        return (k + 1, jnp.concatenate(m_new, axis=0),
                jnp.concatenate(t8_new, axis=0))

    m0 = jnp.max(sim_s[...], axis=1, keepdims=True)
    lax.while_loop(cond, body, (jnp.int32(0), m0, tv_s[:, _K - 1:]))

    @pl.when(j == _NB - 1)
    def _():
        ti_ref[...] = ti_s[...]


def _topk_indices(query_embeddings, keys):
    return pl.pallas_call(
        _topk_body,
        grid=(_NB,),
        in_specs=[
            pl.BlockSpec((_Q, _D), lambda j: (0, 0)),
            pl.BlockSpec((_BM, _D), lambda j: (j, 0)),
        ],
        out_specs=pl.BlockSpec((_Q, _K), lambda j: (0, 0)),
        out_shape=jax.ShapeDtypeStruct((_Q, _K), jnp.int32),
        scratch_shapes=[
            pltpu.VMEM((_Q, _K), jnp.float32),
            pltpu.VMEM((_Q, _K), jnp.int32),
            pltpu.VMEM((_Q, _D), jnp.float32),
            pltpu.VMEM((_Q, _BM), jnp.float32),
        ],
    )(query_embeddings, keys)


def _gather_rows(values, idx_flat):
    info = plsc.get_sparse_core_info()
    nw = info.num_cores * info.num_subcores
    b = idx_flat.shape[0]
    bpw = b // nw
    mesh = plsc.VectorSubcoreMesh(core_axis_name="c", subcore_axis_name="s")

    @functools.partial(
        pl.kernel,
        out_type=jax.ShapeDtypeStruct((b, _D), jnp.float32),
        mesh=mesh,
        scratch_types=[
            pltpu.VMEM((bpw,), jnp.int32),
            pltpu.VMEM((bpw, _D), jnp.float32),
            pltpu.SemaphoreType.DMA,
        ],
    )
    def gather(values_hbm, idx_hbm, out_hbm, idx_v, rows_v, sem):
        wid = lax.axis_index("s") * info.num_cores + lax.axis_index("c")
        base = wid * bpw
        pltpu.sync_copy(idx_hbm.at[pl.ds(base, bpw)], idx_v)
        for c in range(bpw // 128):
            pltpu.async_copy(
                values_hbm.at[idx_v.at[pl.ds(c * 128, 128)]],
                rows_v.at[pl.ds(c * 128, 128)],
                sem,
            ).wait()
        pltpu.sync_copy(rows_v, out_hbm.at[pl.ds(base, bpw)])

    return gather(values, idx_flat)


def kernel(query_embeddings, keys, values, top_k):
    del top_k
    ti = _topk_indices(query_embeddings, keys)
    rows = _gather_rows(values, ti.reshape(-1))
    return rows.reshape(_Q, _K, _D)


# speedup vs baseline: 1.1883x; 1.1883x over previous
import functools

import jax
import jax.numpy as jnp
from jax import lax
from jax.experimental import pallas as pl
from jax.experimental.pallas import tpu as pltpu
from jax.experimental.pallas import tpu_sc as plsc

_Q = 1024
_MEM = 100000
_D = 128
_K = 8
_BM = 2560
_NB = (_MEM + _BM - 1) // _BM
_NEG = float("-inf")


def _topk_body(q_ref, k_ref, ti_ref, tv_s, ti_s, qn_s, sim_s):
    j = pl.program_id(0)

    @pl.when(j == 0)
    def _():
        tv_s[...] = jnp.full((_Q, _K), _NEG, jnp.float32)
        ti_s[...] = jnp.zeros((_Q, _K), jnp.int32)
        q = q_ref[...]
        qn_s[...] = q / jnp.maximum(
            jnp.sqrt(jnp.sum(q * q, axis=1, keepdims=True)), 1e-12)

    kb = k_ref[...]
    kn = kb / jnp.maximum(
        jnp.sqrt(jnp.sum(kb * kb, axis=1, keepdims=True)), 1e-12)
    sim = lax.dot_general(qn_s[...], kn, (((1,), (1,)), ((), ())),
                          preferred_element_type=jnp.float32)
    col = lax.broadcasted_iota(jnp.int32, (_Q, _BM), 1) + j * _BM

    @pl.when(j == _NB - 1)
    def _():
        sim_s[...] = jnp.where(col < _MEM, sim, _NEG)

    @pl.when(j < _NB - 1)
    def _():
        sim_s[...] = sim

    slot = lax.broadcasted_iota(jnp.int32, (_Q, _K), 1)
    big = jnp.int32(2**31 - 1)

    def cond(carry):
        k, m, t8 = carry
        return jnp.logical_and(k < _K, jnp.any(m > t8))

    def body(carry):
        k, m, _ = carry
        s = sim_s[...]
        im = jnp.min(jnp.where(s == m, col, big), axis=1, keepdims=True)
        masked = jnp.where(col == im, _NEG, s)
        sim_s[...] = masked
        tv = tv_s[...]
        ti = ti_s[...]
        pos = jnp.sum((tv >= m).astype(jnp.int32), axis=1, keepdims=True)
        tv_sh = jnp.concatenate([tv[:, :1], tv[:, :-1]], axis=1)
        ti_sh = jnp.concatenate([ti[:, :1], ti[:, :-1]], axis=1)
        keep = slot < pos
        at = slot == pos
        tv_new = jnp.where(keep, tv, jnp.where(at, m, tv_sh))
        ti_new = jnp.where(keep, ti, jnp.where(at, im, ti_sh))
        tv_s[...] = tv_new
        ti_s[...] = ti_new
        m_new = jnp.max(masked, axis=1, keepdims=True)
        return k + 1, m_new, tv_new[:, _K - 1:]

    m0 = jnp.max(sim_s[...], axis=1, keepdims=True)
    lax.while_loop(cond, body, (jnp.int32(0), m0, tv_s[:, _K - 1:]))

    @pl.when(j == _NB - 1)
    def _():
        ti_ref[...] = ti_s[...]


def _topk_indices(query_embeddings, keys):
    return pl.pallas_call(
        _topk_body,
        grid=(_NB,),
        in_specs=[
            pl.BlockSpec((_Q, _D), lambda j: (0, 0)),
            pl.BlockSpec((_BM, _D), lambda j: (j, 0)),
        ],
        out_specs=pl.BlockSpec((_Q, _K), lambda j: (0, 0)),
        out_shape=jax.ShapeDtypeStruct((_Q, _K), jnp.int32),
        scratch_shapes=[
            pltpu.VMEM((_Q, _K), jnp.float32),
            pltpu.VMEM((_Q, _K), jnp.int32),
            pltpu.VMEM((_Q, _D), jnp.float32),
            pltpu.VMEM((_Q, _BM), jnp.float32),
        ],
    )(query_embeddings, keys)


def _gather_rows(values, idx_flat):
    info = plsc.get_sparse_core_info()
    nw = info.num_cores * info.num_subcores
    b = idx_flat.shape[0]
    bpw = b // nw
    mesh = plsc.VectorSubcoreMesh(core_axis_name="c", subcore_axis_name="s")

    @functools.partial(
        pl.kernel,
        out_type=jax.ShapeDtypeStruct((b, _D), jnp.float32),
        mesh=mesh,
        scratch_types=[
            pltpu.VMEM((bpw,), jnp.int32),
            pltpu.VMEM((bpw, _D), jnp.float32),
            pltpu.SemaphoreType.DMA,
        ],
    )
    def gather(values_hbm, idx_hbm, out_hbm, idx_v, rows_v, sem):
        wid = lax.axis_index("s") * info.num_cores + lax.axis_index("c")
        base = wid * bpw
        pltpu.sync_copy(idx_hbm.at[pl.ds(base, bpw)], idx_v)
        for c in range(bpw // 128):
            pltpu.async_copy(
                values_hbm.at[idx_v.at[pl.ds(c * 128, 128)]],
                rows_v.at[pl.ds(c * 128, 128)],
                sem,
            ).wait()
        pltpu.sync_copy(rows_v, out_hbm.at[pl.ds(base, bpw)])

    return gather(values, idx_flat)


def kernel(query_embeddings, keys, values, top_k):
    del top_k
    ti = _topk_indices(query_embeddings, keys)
    rows = _gather_rows(values, ti.reshape(-1))
    return rows.reshape(_Q, _K, _D)


# speedup vs baseline: 1.2034x; 1.0127x over previous
import functools

import jax
import jax.numpy as jnp
from jax import lax
from jax.experimental import pallas as pl
from jax.experimental.pallas import tpu as pltpu
from jax.experimental.pallas import tpu_sc as plsc

_Q = 1024
_MEM = 100000
_D = 128
_K = 8
_BM = 2048
_NB = (_MEM + _BM - 1) // _BM
_NEG = float("-inf")


def _topk_body(q_ref, k_ref, ti_ref, tv_s, ti_s, qn_s, sim_s):
    j = pl.program_id(0)

    @pl.when(j == 0)
    def _():
        tv_s[...] = jnp.full((_Q, _K), _NEG, jnp.float32)
        ti_s[...] = jnp.zeros((_Q, _K), jnp.int32)
        q = q_ref[...]
        qn_s[...] = q / jnp.maximum(
            jnp.sqrt(jnp.sum(q * q, axis=1, keepdims=True)), 1e-12)

    kb = k_ref[...]
    kn = kb / jnp.maximum(
        jnp.sqrt(jnp.sum(kb * kb, axis=1, keepdims=True)), 1e-12)
    sim = lax.dot_general(qn_s[...], kn, (((1,), (1,)), ((), ())),
                          preferred_element_type=jnp.float32)
    col = lax.broadcasted_iota(jnp.int32, (_Q, _BM), 1) + j * _BM

    @pl.when(j == _NB - 1)
    def _():
        sim_s[...] = jnp.where(col < _MEM, sim, _NEG)

    @pl.when(j < _NB - 1)
    def _():
        sim_s[...] = sim

    slot = lax.broadcasted_iota(jnp.int32, (_Q, _K), 1)
    big = jnp.int32(2**31 - 1)

    def cond(carry):
        k, m, t8 = carry
        return jnp.logical_and(k < _K, jnp.any(m > t8))

    def body(carry):
        k, m, _ = carry
        s = sim_s[...]
        im = jnp.min(jnp.where(s == m, col, big), axis=1, keepdims=True)
        masked = jnp.where(col == im, _NEG, s)
        sim_s[...] = masked
        tv = tv_s[...]
        ti = ti_s[...]
        pos = jnp.sum((tv >= m).astype(jnp.int32), axis=1, keepdims=True)
        tv_sh = jnp.concatenate([tv[:, :1], tv[:, :-1]], axis=1)
        ti_sh = jnp.concatenate([ti[:, :1], ti[:, :-1]], axis=1)
        keep = slot < pos
        at = slot == pos
        tv_new = jnp.where(keep, tv, jnp.where(at, m, tv_sh))
        ti_new = jnp.where(keep, ti, jnp.where(at, im, ti_sh))
        tv_s[...] = tv_new
        ti_s[...] = ti_new
        m_new = jnp.max(masked, axis=1, keepdims=True)
        return k + 1, m_new, tv_new[:, _K - 1:]

    m0 = jnp.max(sim_s[...], axis=1, keepdims=True)
    lax.while_loop(cond, body, (jnp.int32(0), m0, tv_s[:, _K - 1:]))

    @pl.when(j == _NB - 1)
    def _():
        ti_ref[...] = ti_s[...]


def _topk_indices(query_embeddings, keys):
    return pl.pallas_call(
        _topk_body,
        grid=(_NB,),
        in_specs=[
            pl.BlockSpec((_Q, _D), lambda j: (0, 0)),
            pl.BlockSpec((_BM, _D), lambda j: (j, 0)),
        ],
        out_specs=pl.BlockSpec((_Q, _K), lambda j: (0, 0)),
        out_shape=jax.ShapeDtypeStruct((_Q, _K), jnp.int32),
        scratch_shapes=[
            pltpu.VMEM((_Q, _K), jnp.float32),
            pltpu.VMEM((_Q, _K), jnp.int32),
            pltpu.VMEM((_Q, _D), jnp.float32),
            pltpu.VMEM((_Q, _BM), jnp.float32),
        ],
    )(query_embeddings, keys)


def _gather_rows(values, idx_flat):
    info = plsc.get_sparse_core_info()
    nw = info.num_cores * info.num_subcores
    b = idx_flat.shape[0]
    bpw = b // nw
    mesh = plsc.VectorSubcoreMesh(core_axis_name="c", subcore_axis_name="s")

    @functools.partial(
        pl.kernel,
        out_type=jax.ShapeDtypeStruct((b, _D), jnp.float32),
        mesh=mesh,
        scratch_types=[
            pltpu.VMEM((bpw,), jnp.int32),
            pltpu.VMEM((bpw, _D), jnp.float32),
            pltpu.SemaphoreType.DMA,
        ],
    )
    def gather(values_hbm, idx_hbm, out_hbm, idx_v, rows_v, sem):
        wid = lax.axis_index("s") * info.num_cores + lax.axis_index("c")
        base = wid * bpw
        pltpu.sync_copy(idx_hbm.at[pl.ds(base, bpw)], idx_v)
        for c in range(bpw // 128):
            pltpu.async_copy(
                values_hbm.at[idx_v.at[pl.ds(c * 128, 128)]],
                rows_v.at[pl.ds(c * 128, 128)],
                sem,
            ).wait()
        pltpu.sync_copy(rows_v, out_hbm.at[pl.ds(base, bpw)])

    return gather(values, idx_flat)


def kernel(query_embeddings, keys, values, top_k):
    del top_k
    ti = _topk_indices(query_embeddings, keys)
    rows = _gather_rows(values, ti.reshape(-1))
    return rows.reshape(_Q, _K, _D)


# speedup vs baseline: 1.2561x; 1.0437x over previous
import functools

import jax
import jax.numpy as jnp
from jax import lax
from jax.experimental import pallas as pl
from jax.experimental.pallas import tpu as pltpu
from jax.experimental.pallas import tpu_sc as plsc

_Q = 1024
_MEM = 100000
_D = 128
_K = 8
_BM = 2048
_NB = (_MEM + _BM - 1) // _BM
_NEG = float("-inf")


def _topk_body(q_ref, k_ref, ti_ref, tv_s, ti_s, qn_s, sim_s):
    j = pl.program_id(0)

    @pl.when(j == 0)
    def _():
        tv_s[...] = jnp.full((_Q, _K), _NEG, jnp.float32)
        ti_s[...] = jnp.zeros((_Q, _K), jnp.int32)
        q = q_ref[...]
        qn_s[...] = q / jnp.maximum(
            jnp.sqrt(jnp.sum(q * q, axis=1, keepdims=True)), 1e-12)

    @pl.when(j < _NB)
    def _():
        kb = k_ref[...]
        kn = kb / jnp.maximum(
            jnp.sqrt(jnp.sum(kb * kb, axis=1, keepdims=True)), 1e-12)
        sim = lax.dot_general(qn_s[...], kn, (((1,), (1,)), ((), ())),
                              preferred_element_type=jnp.float32)
        colj = lax.broadcasted_iota(jnp.int32, (_Q, _BM), 1) + j * _BM
        sim = jnp.where(colj < _MEM, sim, _NEG)
        sim_s[pl.ds(lax.rem(j, 2), 1), :, :] = sim[None]

    @pl.when(j > 0)
    def _():
        jj = j - 1
        col = lax.broadcasted_iota(jnp.int32, (_Q, _BM), 1) + jj * _BM
        pm = lax.rem(jj, 2)
        slot = lax.broadcasted_iota(jnp.int32, (_Q, _K), 1)
        big = jnp.int32(2**31 - 1)

        def cond(carry):
            k, m, t8 = carry
            return jnp.logical_and(k < _K, jnp.any(m > t8))

        def body(carry):
            k, m, _ = carry
            s = sim_s[pl.ds(pm, 1), :, :][0]
            im = jnp.min(jnp.where(s == m, col, big), axis=1,
                         keepdims=True)
            masked = jnp.where(col == im, _NEG, s)
            sim_s[pl.ds(pm, 1), :, :] = masked[None]
            tv = tv_s[...]
            ti = ti_s[...]
            pos = jnp.sum((tv >= m).astype(jnp.int32), axis=1,
                          keepdims=True)
            tv_sh = jnp.concatenate([tv[:, :1], tv[:, :-1]], axis=1)
            ti_sh = jnp.concatenate([ti[:, :1], ti[:, :-1]], axis=1)
            keep = slot < pos
            at = slot == pos
            tv_new = jnp.where(keep, tv, jnp.where(at, m, tv_sh))
            ti_new = jnp.where(keep, ti, jnp.where(at, im, ti_sh))
            tv_s[...] = tv_new
            ti_s[...] = ti_new
            m_new = jnp.max(masked, axis=1, keepdims=True)
            return k + 1, m_new, tv_new[:, _K - 1:]

        m0 = jnp.max(sim_s[pl.ds(pm, 1), :, :][0], axis=1, keepdims=True)
        lax.while_loop(cond, body, (jnp.int32(0), m0, tv_s[:, _K - 1:]))

    @pl.when(j == _NB)
    def _():
        ti_ref[...] = ti_s[...]


def _topk_indices(query_embeddings, keys):
    return pl.pallas_call(
        _topk_body,
        grid=(_NB + 1,),
        in_specs=[
            pl.BlockSpec((_Q, _D), lambda j: (0, 0)),
            pl.BlockSpec((_BM, _D), lambda j: (jnp.minimum(j, _NB - 1), 0)),
        ],
        out_specs=pl.BlockSpec((_Q, _K), lambda j: (0, 0)),
        out_shape=jax.ShapeDtypeStruct((_Q, _K), jnp.int32),
        scratch_shapes=[
            pltpu.VMEM((_Q, _K), jnp.float32),
            pltpu.VMEM((_Q, _K), jnp.int32),
            pltpu.VMEM((_Q, _D), jnp.float32),
            pltpu.VMEM((2, _Q, _BM), jnp.float32),
        ],
    )(query_embeddings, keys)


def _gather_rows(values, idx_flat):
    info = plsc.get_sparse_core_info()
    nw = info.num_cores * info.num_subcores
    b = idx_flat.shape[0]
    bpw = b // nw
    mesh = plsc.VectorSubcoreMesh(core_axis_name="c", subcore_axis_name="s")

    @functools.partial(
        pl.kernel,
        out_type=jax.ShapeDtypeStruct((b, _D), jnp.float32),
        mesh=mesh,
        scratch_types=[
            pltpu.VMEM((bpw,), jnp.int32),
            pltpu.VMEM((bpw, _D), jnp.float32),
            pltpu.SemaphoreType.DMA,
        ],
    )
    def gather(values_hbm, idx_hbm, out_hbm, idx_v, rows_v, sem):
        wid = lax.axis_index("s") * info.num_cores + lax.axis_index("c")
        base = wid * bpw
        pltpu.sync_copy(idx_hbm.at[pl.ds(base, bpw)], idx_v)
        for c in range(bpw // 128):
            pltpu.async_copy(
                values_hbm.at[idx_v.at[pl.ds(c * 128, 128)]],
                rows_v.at[pl.ds(c * 128, 128)],
                sem,
            ).wait()
        pltpu.sync_copy(rows_v, out_hbm.at[pl.ds(base, bpw)])

    return gather(values, idx_flat)


def kernel(query_embeddings, keys, values, top_k):
    del top_k
    ti = _topk_indices(query_embeddings, keys)
    rows = _gather_rows(values, ti.reshape(-1))
    return rows.reshape(_Q, _K, _D)
